# baseline (speedup 1.0000x reference)
import jax
import jax.numpy as jnp
from jax.experimental import pallas as pl

_NUM_EXPERTS = 8
_EPAD = 256
_TILE = 512


def _moe_body(x_ref, wg_ref, bg_ref, w2_ref, be_ref, o_ref):
    xt = x_ref[...]
    logits = jnp.dot(xt, wg_ref[...], preferred_element_type=jnp.float32)
    logits = logits + bg_ref[...]
    g = jax.nn.softmax(logits, axis=-1)
    m1 = jnp.max(g, axis=-1, keepdims=True)
    g_no_top1 = jnp.where(g >= m1, -jnp.inf, g)
    m2 = jnp.max(g_no_top1, axis=-1, keepdims=True)
    wfull = jnp.where(g >= m2, g, 0.0) / (m1 + m2)
    xb = xt.astype(jnp.bfloat16)
    y = jnp.dot(xb, w2_ref[...], preferred_element_type=jnp.float32)
    acc = jnp.dot(wfull, be_ref[...], preferred_element_type=jnp.float32)
    for e in range(_NUM_EXPERTS):
        acc = acc + wfull[:, e:e + 1] * y[:, e * _EPAD:e * _EPAD + _EPAD]
    o_ref[...] = acc[:, :o_ref.shape[1]]


def _forward(x, Wg, bg, We, be, *, interpret=False):
    B, S, D = x.shape
    E = Wg.shape[-1]
    n = B * S
    xf = x.reshape(n, D)
    w2 = jnp.pad(jnp.transpose(We, (1, 0, 2)),
                 ((0, 0), (0, 0), (0, _EPAD - D))).reshape(D, E * _EPAD)
    be_pad = jnp.pad(be, ((0, 0), (0, _EPAD - D)))
    grid = (n // _TILE,)
    out = pl.pallas_call(
        _moe_body,
        grid=grid,
        in_specs=[
            pl.BlockSpec((_TILE, D), lambda i: (i, 0)),
            pl.BlockSpec((D, E), lambda i: (0, 0)),
            pl.BlockSpec((1, E), lambda i: (0, 0)),
            pl.BlockSpec((D, E * _EPAD), lambda i: (0, 0)),
            pl.BlockSpec((E, _EPAD), lambda i: (0, 0)),
        ],
        out_specs=pl.BlockSpec((_TILE, D), lambda i: (i, 0)),
        out_shape=jax.ShapeDtypeStruct((n, D), jnp.float32),
        interpret=interpret,
    )(xf, Wg, bg.reshape(1, E), w2.astype(jnp.bfloat16), be_pad)
    return out.reshape(B, S, D)


def kernel(x, Wg, bg, We, be):
    return _forward(x, Wg, bg, We, be)
